# baseline (speedup 1.0000x reference)
import functools

import jax
import jax.numpy as jnp
from jax import lax
from jax.experimental import pallas as pl
from jax.experimental.pallas import tpu as pltpu
from jax.experimental.pallas import tpu_sc as plsc

N = 10000
E = 320000
DF = 128
EMB = 4096
HID = 32
OUT = 5
RNK = 8

NB = 10
BLK = N // NB

NC = 2
NS = 16
NW = NC * NS
CH = 1000
NCHUNK = 10


@functools.cache
def _get_sc_degree():
    mesh = plsc.VectorSubcoreMesh(core_axis_name="c", subcore_axis_name="s",
                                  num_cores=NC, num_subcores=NS)
    return pl.kernel(
        _sc_degree_body,
        out_type=jax.ShapeDtypeStruct((N,), jnp.float32),
        mesh=mesh,
        scratch_types=[
            pltpu.VMEM((2 * NCHUNK, CH), jnp.int32),
            pltpu.VMEM((CH,), jnp.float32),
            pltpu.VMEM_SHARED((N,), jnp.float32),
            pltpu.SemaphoreType.DMA,
        ],
        compiler_params=pltpu.CompilerParams(use_tc_tiling_on_sc=False),
    )


def _sc_degree_body(dst_hbm, zeros_hbm, ones_hbm, out_hbm, dst_v, ones_v, acc, sem):
    cid = lax.axis_index("c")
    sid = lax.axis_index("s")

    @pl.when((cid == 0) & (sid == 0))
    def _():
        pltpu.sync_copy(zeros_hbm, acc)
    plsc.subcore_barrier()

    @pl.when(cid == 0)
    def _():
        pltpu.sync_copy(ones_hbm, ones_v)
        pltpu.sync_copy(dst_hbm.at[sid], dst_v)

        def body(j, carry):
            pltpu.sync_copy(ones_v, acc.at[dst_v.at[j]], add=True)
            return carry

        lax.fori_loop(0, 2 * NCHUNK, body, 0)

    plsc.subcore_barrier()

    @pl.when((cid == 0) & (sid == 0))
    def _():
        pltpu.sync_copy(acc, out_hbm)


@functools.cache
def _get_sc_scatter():
    mesh = plsc.VectorSubcoreMesh(core_axis_name="c", subcore_axis_name="s",
                                  num_cores=NC, num_subcores=NS)
    return pl.kernel(
        _sc_scatter_body,
        out_type=jax.ShapeDtypeStruct((NC, N, HID), jnp.float32),
        mesh=mesh,
        scratch_types=[
            pltpu.VMEM((NCHUNK, CH), jnp.int32),
            pltpu.VMEM((NCHUNK, CH), jnp.int32),
            pltpu.VMEM((CH, HID), jnp.float32),
            pltpu.VMEM((CH, HID), jnp.float32),
            pltpu.VMEM_SHARED((N, HID), jnp.float32),
            pltpu.SemaphoreType.DMA,
            pltpu.SemaphoreType.DMA,
        ],
        compiler_params=pltpu.CompilerParams(use_tc_tiling_on_sc=False),
    )


def _sc_scatter_body(g_hbm, src_hbm, dst_hbm, out_hbm, src_v, dst_v,
                     rows0, rows1, acc, sem0, sem1):
    cid = lax.axis_index("c")
    sid = lax.axis_index("s")
    wid = cid * NS + sid

    @pl.when(sid == 0)
    def _():
        pltpu.sync_copy(g_hbm, acc)
    plsc.subcore_barrier()

    pltpu.sync_copy(src_hbm.at[wid], src_v)
    pltpu.sync_copy(dst_hbm.at[wid], dst_v)

    pltpu.async_copy(g_hbm.at[src_v.at[0]], rows0, sem0)

    def body(i, carry):
        j = 2 * i
        pltpu.async_copy(g_hbm.at[src_v.at[j + 1]], rows1, sem1)
        pltpu.make_async_copy(g_hbm.at[src_v.at[j]], rows0, sem0).wait()
        pltpu.sync_copy(rows0, acc.at[dst_v.at[j]], add=True)

        @pl.when(i < NCHUNK // 2 - 1)
        def _():
            pltpu.async_copy(g_hbm.at[src_v.at[j + 2]], rows0, sem0)

        pltpu.make_async_copy(g_hbm.at[src_v.at[j + 1]], rows1, sem1).wait()
        pltpu.sync_copy(rows1, acc.at[dst_v.at[j + 1]], add=True)
        return carry

    lax.fori_loop(0, NCHUNK // 2, body, 0)
    plsc.subcore_barrier()

    @pl.when(sid == 0)
    def _():
        pltpu.sync_copy(acc, out_hbm.at[cid])


def _tc_a_body(x_ref, didx_ref, cnt_ref, emb_ref, w1e_ref, w1x_ref,
               g1_ref, dinv_ref):
    embw = jnp.dot(emb_ref[...], w1e_ref[...],
                   preferred_element_type=jnp.float32)
    d = didx_ref[0]
    e0 = embw[0:1, :]
    e1 = embw[1:2, :]
    e2 = embw[2:3, :]
    embedded = jnp.where(d == 0, e0, jnp.where(d == 1, e1, e2))
    hx = jnp.dot(x_ref[...], w1x_ref[...],
                 preferred_element_type=jnp.float32)
    dinv = lax.rsqrt(cnt_ref[...] + 1.0)
    g1_ref[...] = dinv * (embedded + hx)
    dinv_ref[...] = dinv


def _tc_a(x, didx3, cnt2, emb_pad, w1e, w1x):
    return pl.pallas_call(
        _tc_a_body,
        grid=(NB,),
        in_specs=[
            pl.BlockSpec((BLK, DF), lambda i: (i, 0)),
            pl.BlockSpec((1, BLK, 1), lambda i: (i, 0, 0)),
            pl.BlockSpec((BLK, 1), lambda i: (i, 0)),
            pl.BlockSpec((8, EMB), lambda i: (0, 0)),
            pl.BlockSpec((EMB, HID), lambda i: (0, 0)),
            pl.BlockSpec((DF, HID), lambda i: (0, 0)),
        ],
        out_specs=[
            pl.BlockSpec((BLK, HID), lambda i: (i, 0)),
            pl.BlockSpec((BLK, 1), lambda i: (i, 0)),
        ],
        out_shape=[
            jax.ShapeDtypeStruct((N, HID), jnp.float32),
            jax.ShapeDtypeStruct((N, 1), jnp.float32),
        ],
    )(x, didx3, cnt2, emb_pad, w1e, w1x)


def _tc_b_body(p0_ref, p1_ref, g1_ref, dinv_ref, b1_ref, a1_ref, b1m_ref,
               g2_ref):
    dinv = dinv_ref[...]
    s = (p0_ref[...] + p1_ref[...] - g1_ref[...]) * dinv + b1_ref[...]
    m1 = jnp.dot(a1_ref[...], b1m_ref[...],
                 preferred_element_type=jnp.float32) * (1.0 / RNK)
    h = jnp.maximum(jnp.dot(s, m1, preferred_element_type=jnp.float32), 0.0)
    g2_ref[...] = dinv * h


def _tc_b(p0, p1, g1, dinv, b1r, A1, B1):
    return pl.pallas_call(
        _tc_b_body,
        grid=(NB,),
        in_specs=[
            pl.BlockSpec((BLK, HID), lambda i: (i, 0)),
            pl.BlockSpec((BLK, HID), lambda i: (i, 0)),
            pl.BlockSpec((BLK, HID), lambda i: (i, 0)),
            pl.BlockSpec((BLK, 1), lambda i: (i, 0)),
            pl.BlockSpec((1, HID), lambda i: (0, 0)),
            pl.BlockSpec((HID, RNK), lambda i: (0, 0)),
            pl.BlockSpec((RNK, HID), lambda i: (0, 0)),
        ],
        out_specs=pl.BlockSpec((BLK, HID), lambda i: (i, 0)),
        out_shape=jax.ShapeDtypeStruct((N, HID), jnp.float32),
    )(p0, p1, g1, dinv, b1r, A1, B1)


def _tc_c_body(q0_ref, q1_ref, g2_ref, dinv_ref, w2_ref, b2_ref, a2_ref,
               b2m_ref, out_ref):
    s2 = (q0_ref[...] + q1_ref[...] - g2_ref[...]) * dinv_ref[...]
    ab2 = jnp.dot(a2_ref[...], b2m_ref[...],
                  preferred_element_type=jnp.float32) * (1.0 / RNK)
    m2 = jnp.dot(w2_ref[...], ab2, preferred_element_type=jnp.float32)
    c2 = jnp.dot(b2_ref[...], ab2, preferred_element_type=jnp.float32)
    y = jnp.dot(s2, m2, preferred_element_type=jnp.float32) + c2
    col = lax.broadcasted_iota(jnp.int32, (BLK, 8), 1)
    y = jnp.where(col < OUT, y, -1e30)
    m = jnp.max(y, axis=1, keepdims=True)
    z = y - m
    lse = jnp.log(jnp.sum(jnp.exp(z), axis=1, keepdims=True))
    out_ref[...] = z - lse


def _tc_c(q0, q1, g2, dinv, W2p, b2p, A2p, B2p):
    return pl.pallas_call(
        _tc_c_body,
        grid=(NB,),
        in_specs=[
            pl.BlockSpec((BLK, HID), lambda i: (i, 0)),
            pl.BlockSpec((BLK, HID), lambda i: (i, 0)),
            pl.BlockSpec((BLK, HID), lambda i: (i, 0)),
            pl.BlockSpec((BLK, 1), lambda i: (i, 0)),
            pl.BlockSpec((HID, 8), lambda i: (0, 0)),
            pl.BlockSpec((1, 8), lambda i: (0, 0)),
            pl.BlockSpec((8, 8), lambda i: (0, 0)),
            pl.BlockSpec((8, 8), lambda i: (0, 0)),
        ],
        out_specs=pl.BlockSpec((BLK, 8), lambda i: (i, 0)),
        out_shape=jax.ShapeDtypeStruct((N, 8), jnp.float32),
    )(q0, q1, g2, dinv, W2p, b2p, A2p, B2p)


def kernel(x, edge_index, domain_idx, emb, W1, b1, A1, B1, W2, b2, A2, B2):
    f32 = jnp.float32
    ei = edge_index.reshape(2, NW, NCHUNK, CH)
    src_p = ei[0]
    dst_p = ei[1]
    dst_deg = edge_index[1].reshape(NS, 2 * NCHUNK, CH)

    cnt = _get_sc_degree()(dst_deg, jnp.zeros((N,), f32), jnp.ones((CH,), f32))
    cnt2 = cnt.reshape(N, 1)

    didx3 = domain_idx.reshape(NB, BLK, 1)
    emb_pad = jnp.zeros((8, EMB), f32).at[:3].set(emb)
    w1e = W1[:EMB]
    w1x = W1[EMB:]

    g1, dinv = _tc_a(x, didx3, cnt2, emb_pad, w1e, w1x)

    p = _get_sc_scatter()(g1, src_p, dst_p)

    g2 = _tc_b(p[0], p[1], g1, dinv, b1.reshape(1, HID), A1, B1)

    q = _get_sc_scatter()(g2, src_p, dst_p)

    W2p = jnp.zeros((HID, 8), f32).at[:, :OUT].set(W2)
    b2p = jnp.zeros((1, 8), f32).at[0, :OUT].set(b2)
    A2p = jnp.zeros((8, 8), f32).at[:OUT].set(A2)
    B2p = jnp.zeros((8, 8), f32).at[:, :OUT].set(B2)

    out = _tc_c(q[0], q[1], g2, dinv, W2p, b2p, A2p, B2p)
    return out[:, :OUT]


# speedup vs baseline: 1.0014x; 1.0014x over previous
import functools

import jax
import jax.numpy as jnp
from jax import lax
from jax.experimental import pallas as pl
from jax.experimental.pallas import tpu as pltpu
from jax.experimental.pallas import tpu_sc as plsc

N = 10000
E = 320000
DF = 128
EMB = 4096
HID = 32
OUT = 5
RNK = 8

NB = 10
BLK = N // NB

NC = 2
NS = 16
NW = NC * NS
CH = 1000
NCHUNK = 10


@functools.cache
def _get_sc_degree():
    mesh = plsc.VectorSubcoreMesh(core_axis_name="c", subcore_axis_name="s",
                                  num_cores=NC, num_subcores=NS)
    return pl.kernel(
        _sc_degree_body,
        out_type=jax.ShapeDtypeStruct((N,), jnp.float32),
        mesh=mesh,
        scratch_types=[
            pltpu.VMEM((2 * NCHUNK, CH), jnp.int32),
            pltpu.VMEM((CH,), jnp.float32),
            pltpu.VMEM_SHARED((N,), jnp.float32),
            pltpu.SemaphoreType.DMA,
        ],
        compiler_params=pltpu.CompilerParams(use_tc_tiling_on_sc=False),
    )


def _sc_degree_body(dst_hbm, zeros_hbm, ones_hbm, out_hbm, dst_v, ones_v, acc, sem):
    cid = lax.axis_index("c")
    sid = lax.axis_index("s")

    @pl.when((cid == 0) & (sid == 0))
    def _():
        pltpu.sync_copy(zeros_hbm, acc)
    plsc.subcore_barrier()

    @pl.when(cid == 0)
    def _():
        pltpu.sync_copy(ones_hbm, ones_v)
        pltpu.sync_copy(dst_hbm.at[sid], dst_v)

        def body(j, carry):
            pltpu.sync_copy(ones_v, acc.at[dst_v.at[j]], add=True)
            return carry

        lax.fori_loop(0, 2 * NCHUNK, body, 0)

    plsc.subcore_barrier()

    @pl.when((cid == 0) & (sid == 0))
    def _():
        pltpu.sync_copy(acc, out_hbm)


@functools.cache
def _get_sc_scatter():
    mesh = plsc.VectorSubcoreMesh(core_axis_name="c", subcore_axis_name="s",
                                  num_cores=NC, num_subcores=NS)
    return pl.kernel(
        _sc_scatter_body,
        out_type=jax.ShapeDtypeStruct((NC, N, HID), jnp.float32),
        mesh=mesh,
        scratch_types=[
            pltpu.VMEM((NCHUNK, CH), jnp.int32),
            pltpu.VMEM((NCHUNK, CH), jnp.int32),
            pltpu.VMEM((CH, HID), jnp.float32),
            pltpu.VMEM((CH, HID), jnp.float32),
            pltpu.VMEM_SHARED((N, HID), jnp.float32),
            pltpu.SemaphoreType.DMA,
            pltpu.SemaphoreType.DMA,
        ],
        compiler_params=pltpu.CompilerParams(use_tc_tiling_on_sc=False),
    )


def _sc_scatter_body(g_hbm, src_hbm, dst_hbm, out_hbm, src_v, dst_v,
                     rows0, rows1, acc, sem0, sem1):
    cid = lax.axis_index("c")
    sid = lax.axis_index("s")
    wid = cid * NS + sid

    @pl.when(sid == 0)
    def _():
        pltpu.sync_copy(g_hbm, acc)
    plsc.subcore_barrier()

    pltpu.sync_copy(src_hbm.at[wid], src_v)
    pltpu.sync_copy(dst_hbm.at[wid], dst_v)

    pltpu.async_copy(g_hbm.at[src_v.at[0]], rows0, sem0)

    def body(i, carry):
        j = 2 * i
        pltpu.async_copy(g_hbm.at[src_v.at[j + 1]], rows1, sem1)
        pltpu.make_async_copy(g_hbm.at[src_v.at[j]], rows0, sem0).wait()
        pltpu.sync_copy(rows0, acc.at[dst_v.at[j]], add=True)

        @pl.when(i < NCHUNK // 2 - 1)
        def _():
            pltpu.async_copy(g_hbm.at[src_v.at[j + 2]], rows0, sem0)

        pltpu.make_async_copy(g_hbm.at[src_v.at[j + 1]], rows1, sem1).wait()
        pltpu.sync_copy(rows1, acc.at[dst_v.at[j + 1]], add=True)
        return carry

    lax.fori_loop(0, NCHUNK // 2, body, 0)
    plsc.subcore_barrier()

    @pl.when(sid == 0)
    def _():
        pltpu.sync_copy(acc, out_hbm.at[cid])


def _tc_a_body(x_ref, didx_ref, cnt_ref, emb_ref, w1e_ref, w1x_ref,
               g1_ref, dinv_ref):
    embw = jnp.dot(emb_ref[...], w1e_ref[...],
                   preferred_element_type=jnp.float32)
    d = didx_ref[0]
    e0 = embw[0:1, :]
    e1 = embw[1:2, :]
    e2 = embw[2:3, :]
    embedded = jnp.where(d == 0, e0, jnp.where(d == 1, e1, e2))
    hx = jnp.dot(x_ref[...], w1x_ref[...],
                 preferred_element_type=jnp.float32)
    dinv = lax.rsqrt(cnt_ref[...] + 1.0)
    g1_ref[...] = dinv * (embedded + hx)
    dinv_ref[...] = dinv


def _tc_a(x, didx3, cnt2, emb_raw, w1e, w1x):
    return pl.pallas_call(
        _tc_a_body,
        grid=(NB,),
        in_specs=[
            pl.BlockSpec((BLK, DF), lambda i: (i, 0)),
            pl.BlockSpec((1, BLK, 1), lambda i: (i, 0, 0)),
            pl.BlockSpec((BLK, 1), lambda i: (i, 0)),
            pl.BlockSpec((3, EMB), lambda i: (0, 0)),
            pl.BlockSpec((EMB, HID), lambda i: (0, 0)),
            pl.BlockSpec((DF, HID), lambda i: (0, 0)),
        ],
        out_specs=[
            pl.BlockSpec((BLK, HID), lambda i: (i, 0)),
            pl.BlockSpec((BLK, 1), lambda i: (i, 0)),
        ],
        out_shape=[
            jax.ShapeDtypeStruct((N, HID), jnp.float32),
            jax.ShapeDtypeStruct((N, 1), jnp.float32),
        ],
    )(x, didx3, cnt2, emb_raw, w1e, w1x)


def _tc_b_body(p0_ref, p1_ref, g1_ref, dinv_ref, b1_ref, a1_ref, b1m_ref,
               g2_ref):
    dinv = dinv_ref[...]
    s = (p0_ref[...] + p1_ref[...] - g1_ref[...]) * dinv + b1_ref[...]
    m1 = jnp.dot(a1_ref[...], b1m_ref[...],
                 preferred_element_type=jnp.float32) * (1.0 / RNK)
    h = jnp.maximum(jnp.dot(s, m1, preferred_element_type=jnp.float32), 0.0)
    g2_ref[...] = dinv * h


def _tc_b(p0, p1, g1, dinv, b1r, A1, B1):
    return pl.pallas_call(
        _tc_b_body,
        grid=(NB,),
        in_specs=[
            pl.BlockSpec((BLK, HID), lambda i: (i, 0)),
            pl.BlockSpec((BLK, HID), lambda i: (i, 0)),
            pl.BlockSpec((BLK, HID), lambda i: (i, 0)),
            pl.BlockSpec((BLK, 1), lambda i: (i, 0)),
            pl.BlockSpec((1, HID), lambda i: (0, 0)),
            pl.BlockSpec((HID, RNK), lambda i: (0, 0)),
            pl.BlockSpec((RNK, HID), lambda i: (0, 0)),
        ],
        out_specs=pl.BlockSpec((BLK, HID), lambda i: (i, 0)),
        out_shape=jax.ShapeDtypeStruct((N, HID), jnp.float32),
    )(p0, p1, g1, dinv, b1r, A1, B1)


def _tc_c_body(q0_ref, q1_ref, g2_ref, dinv_ref, w2_ref, b2_ref, a2_ref,
               b2m_ref, out_ref):
    s2 = (q0_ref[...] + q1_ref[...] - g2_ref[...]) * dinv_ref[...]
    ab2 = jnp.dot(a2_ref[...], b2m_ref[...],
                  preferred_element_type=jnp.float32) * (1.0 / RNK)
    m2 = jnp.dot(w2_ref[...], ab2, preferred_element_type=jnp.float32)
    c2 = jnp.dot(b2_ref[...], ab2, preferred_element_type=jnp.float32)
    y = jnp.dot(s2, m2, preferred_element_type=jnp.float32) + c2
    m = jnp.max(y, axis=1, keepdims=True)
    z = y - m
    lse = jnp.log(jnp.sum(jnp.exp(z), axis=1, keepdims=True))
    out_ref[...] = z - lse


def _tc_c(q0, q1, g2, dinv, W2, b2r, A2, B2):
    return pl.pallas_call(
        _tc_c_body,
        grid=(NB,),
        in_specs=[
            pl.BlockSpec((BLK, HID), lambda i: (i, 0)),
            pl.BlockSpec((BLK, HID), lambda i: (i, 0)),
            pl.BlockSpec((BLK, HID), lambda i: (i, 0)),
            pl.BlockSpec((BLK, 1), lambda i: (i, 0)),
            pl.BlockSpec((HID, OUT), lambda i: (0, 0)),
            pl.BlockSpec((1, OUT), lambda i: (0, 0)),
            pl.BlockSpec((OUT, RNK), lambda i: (0, 0)),
            pl.BlockSpec((RNK, OUT), lambda i: (0, 0)),
        ],
        out_specs=pl.BlockSpec((BLK, OUT), lambda i: (i, 0)),
        out_shape=jax.ShapeDtypeStruct((N, OUT), jnp.float32),
    )(q0, q1, g2, dinv, W2, b2r, A2, B2)


def kernel(x, edge_index, domain_idx, emb, W1, b1, A1, B1, W2, b2, A2, B2):
    f32 = jnp.float32
    ei = edge_index.reshape(2, NW, NCHUNK, CH)
    src_p = ei[0]
    dst_p = ei[1]
    dst_deg = edge_index[1].reshape(NS, 2 * NCHUNK, CH)

    cnt = _get_sc_degree()(dst_deg, jnp.zeros((N,), f32), jnp.ones((CH,), f32))
    cnt2 = cnt.reshape(N, 1)

    didx3 = domain_idx.reshape(NB, BLK, 1)
    w1e = W1[:EMB]
    w1x = W1[EMB:]

    g1, dinv = _tc_a(x, didx3, cnt2, emb, w1e, w1x)

    p = _get_sc_scatter()(g1, src_p, dst_p)

    g2 = _tc_b(p[0], p[1], g1, dinv, b1.reshape(1, HID), A1, B1)

    q = _get_sc_scatter()(g2, src_p, dst_p)

    return _tc_c(q[0], q[1], g2, dinv, W2, b2.reshape(1, OUT), A2, B2)


# speedup vs baseline: 1.0303x; 1.0288x over previous
import functools

import jax
import jax.numpy as jnp
from jax import lax
from jax.experimental import pallas as pl
from jax.experimental.pallas import tpu as pltpu
from jax.experimental.pallas import tpu_sc as plsc

N = 10000
E = 320000
DF = 128
EMB = 4096
HID = 32
OUT = 5
RNK = 8

NB = 1
BLK = N // NB

NC = 2
NS = 16
NW = NC * NS
CH = 1000
NCHUNK = 10


@functools.cache
def _get_sc_degree():
    mesh = plsc.VectorSubcoreMesh(core_axis_name="c", subcore_axis_name="s",
                                  num_cores=NC, num_subcores=NS)
    return pl.kernel(
        _sc_degree_body,
        out_type=jax.ShapeDtypeStruct((N,), jnp.float32),
        mesh=mesh,
        scratch_types=[
            pltpu.VMEM((2 * NCHUNK, CH), jnp.int32),
            pltpu.VMEM((CH,), jnp.float32),
            pltpu.VMEM_SHARED((N,), jnp.float32),
            pltpu.SemaphoreType.DMA,
        ],
        compiler_params=pltpu.CompilerParams(use_tc_tiling_on_sc=False),
    )


def _sc_degree_body(dst_hbm, zeros_hbm, ones_hbm, out_hbm, dst_v, ones_v, acc, sem):
    cid = lax.axis_index("c")
    sid = lax.axis_index("s")

    @pl.when((cid == 0) & (sid == 0))
    def _():
        pltpu.sync_copy(zeros_hbm, acc)
    plsc.subcore_barrier()

    @pl.when(cid == 0)
    def _():
        pltpu.sync_copy(ones_hbm, ones_v)
        pltpu.sync_copy(dst_hbm.at[sid], dst_v)

        def body(j, carry):
            pltpu.sync_copy(ones_v, acc.at[dst_v.at[j]], add=True)
            return carry

        lax.fori_loop(0, 2 * NCHUNK, body, 0)

    plsc.subcore_barrier()

    @pl.when((cid == 0) & (sid == 0))
    def _():
        pltpu.sync_copy(acc, out_hbm)


@functools.cache
def _get_sc_scatter():
    mesh = plsc.VectorSubcoreMesh(core_axis_name="c", subcore_axis_name="s",
                                  num_cores=NC, num_subcores=NS)
    return pl.kernel(
        _sc_scatter_body,
        out_type=jax.ShapeDtypeStruct((NC, N, HID), jnp.float32),
        mesh=mesh,
        scratch_types=[
            pltpu.VMEM((NCHUNK, CH), jnp.int32),
            pltpu.VMEM((NCHUNK, CH), jnp.int32),
            pltpu.VMEM((CH, HID), jnp.float32),
            pltpu.VMEM((CH, HID), jnp.float32),
            pltpu.VMEM_SHARED((N, HID), jnp.float32),
            pltpu.SemaphoreType.DMA,
            pltpu.SemaphoreType.DMA,
        ],
        compiler_params=pltpu.CompilerParams(use_tc_tiling_on_sc=False),
    )


def _sc_scatter_body(g_hbm, src_hbm, dst_hbm, out_hbm, src_v, dst_v,
                     rows0, rows1, acc, sem0, sem1):
    cid = lax.axis_index("c")
    sid = lax.axis_index("s")
    wid = cid * NS + sid

    @pl.when(sid == 0)
    def _():
        pltpu.sync_copy(g_hbm, acc)
    plsc.subcore_barrier()

    pltpu.sync_copy(src_hbm.at[wid], src_v)
    pltpu.sync_copy(dst_hbm.at[wid], dst_v)

    pltpu.async_copy(g_hbm.at[src_v.at[0]], rows0, sem0)

    def body(i, carry):
        j = 2 * i
        pltpu.async_copy(g_hbm.at[src_v.at[j + 1]], rows1, sem1)
        pltpu.make_async_copy(g_hbm.at[src_v.at[j]], rows0, sem0).wait()
        pltpu.sync_copy(rows0, acc.at[dst_v.at[j]], add=True)

        @pl.when(i < NCHUNK // 2 - 1)
        def _():
            pltpu.async_copy(g_hbm.at[src_v.at[j + 2]], rows0, sem0)

        pltpu.make_async_copy(g_hbm.at[src_v.at[j + 1]], rows1, sem1).wait()
        pltpu.sync_copy(rows1, acc.at[dst_v.at[j + 1]], add=True)
        return carry

    lax.fori_loop(0, NCHUNK // 2, body, 0)
    plsc.subcore_barrier()

    @pl.when(sid == 0)
    def _():
        pltpu.sync_copy(acc, out_hbm.at[cid])


def _tc_a_body(x_ref, didx_ref, cnt_ref, emb_ref, w1e_ref, w1x_ref,
               g1_ref, dinv_ref):
    embw = jnp.dot(emb_ref[...], w1e_ref[...],
                   preferred_element_type=jnp.float32)
    d = didx_ref[0]
    e0 = embw[0:1, :]
    e1 = embw[1:2, :]
    e2 = embw[2:3, :]
    embedded = jnp.where(d == 0, e0, jnp.where(d == 1, e1, e2))
    hx = jnp.dot(x_ref[...], w1x_ref[...],
                 preferred_element_type=jnp.float32)
    dinv = lax.rsqrt(cnt_ref[...] + 1.0)
    g1_ref[...] = dinv * (embedded + hx)
    dinv_ref[...] = dinv


def _tc_a(x, didx3, cnt2, emb_raw, w1e, w1x):
    return pl.pallas_call(
        _tc_a_body,
        grid=(NB,),
        in_specs=[
            pl.BlockSpec((BLK, DF), lambda i: (i, 0)),
            pl.BlockSpec((1, BLK, 1), lambda i: (i, 0, 0)),
            pl.BlockSpec((BLK, 1), lambda i: (i, 0)),
            pl.BlockSpec((3, EMB), lambda i: (0, 0)),
            pl.BlockSpec((EMB, HID), lambda i: (0, 0)),
            pl.BlockSpec((DF, HID), lambda i: (0, 0)),
        ],
        out_specs=[
            pl.BlockSpec((BLK, HID), lambda i: (i, 0)),
            pl.BlockSpec((BLK, 1), lambda i: (i, 0)),
        ],
        out_shape=[
            jax.ShapeDtypeStruct((N, HID), jnp.float32),
            jax.ShapeDtypeStruct((N, 1), jnp.float32),
        ],
    )(x, didx3, cnt2, emb_raw, w1e, w1x)


def _tc_b_body(p0_ref, p1_ref, g1_ref, dinv_ref, b1_ref, a1_ref, b1m_ref,
               g2_ref):
    dinv = dinv_ref[...]
    s = (p0_ref[...] + p1_ref[...] - g1_ref[...]) * dinv + b1_ref[...]
    m1 = jnp.dot(a1_ref[...], b1m_ref[...],
                 preferred_element_type=jnp.float32) * (1.0 / RNK)
    h = jnp.maximum(jnp.dot(s, m1, preferred_element_type=jnp.float32), 0.0)
    g2_ref[...] = dinv * h


def _tc_b(p0, p1, g1, dinv, b1r, A1, B1):
    return pl.pallas_call(
        _tc_b_body,
        grid=(NB,),
        in_specs=[
            pl.BlockSpec((BLK, HID), lambda i: (i, 0)),
            pl.BlockSpec((BLK, HID), lambda i: (i, 0)),
            pl.BlockSpec((BLK, HID), lambda i: (i, 0)),
            pl.BlockSpec((BLK, 1), lambda i: (i, 0)),
            pl.BlockSpec((1, HID), lambda i: (0, 0)),
            pl.BlockSpec((HID, RNK), lambda i: (0, 0)),
            pl.BlockSpec((RNK, HID), lambda i: (0, 0)),
        ],
        out_specs=pl.BlockSpec((BLK, HID), lambda i: (i, 0)),
        out_shape=jax.ShapeDtypeStruct((N, HID), jnp.float32),
    )(p0, p1, g1, dinv, b1r, A1, B1)


def _tc_c_body(q0_ref, q1_ref, g2_ref, dinv_ref, w2_ref, b2_ref, a2_ref,
               b2m_ref, out_ref):
    s2 = (q0_ref[...] + q1_ref[...] - g2_ref[...]) * dinv_ref[...]
    ab2 = jnp.dot(a2_ref[...], b2m_ref[...],
                  preferred_element_type=jnp.float32) * (1.0 / RNK)
    m2 = jnp.dot(w2_ref[...], ab2, preferred_element_type=jnp.float32)
    c2 = jnp.dot(b2_ref[...], ab2, preferred_element_type=jnp.float32)
    y = jnp.dot(s2, m2, preferred_element_type=jnp.float32) + c2
    m = jnp.max(y, axis=1, keepdims=True)
    z = y - m
    lse = jnp.log(jnp.sum(jnp.exp(z), axis=1, keepdims=True))
    out_ref[...] = z - lse


def _tc_c(q0, q1, g2, dinv, W2, b2r, A2, B2):
    return pl.pallas_call(
        _tc_c_body,
        grid=(NB,),
        in_specs=[
            pl.BlockSpec((BLK, HID), lambda i: (i, 0)),
            pl.BlockSpec((BLK, HID), lambda i: (i, 0)),
            pl.BlockSpec((BLK, HID), lambda i: (i, 0)),
            pl.BlockSpec((BLK, 1), lambda i: (i, 0)),
            pl.BlockSpec((HID, OUT), lambda i: (0, 0)),
            pl.BlockSpec((1, OUT), lambda i: (0, 0)),
            pl.BlockSpec((OUT, RNK), lambda i: (0, 0)),
            pl.BlockSpec((RNK, OUT), lambda i: (0, 0)),
        ],
        out_specs=pl.BlockSpec((BLK, OUT), lambda i: (i, 0)),
        out_shape=jax.ShapeDtypeStruct((N, OUT), jnp.float32),
    )(q0, q1, g2, dinv, W2, b2r, A2, B2)


def kernel(x, edge_index, domain_idx, emb, W1, b1, A1, B1, W2, b2, A2, B2):
    f32 = jnp.float32
    ei = edge_index.reshape(2, NW, NCHUNK, CH)
    src_p = ei[0]
    dst_p = ei[1]
    dst_deg = edge_index[1].reshape(NS, 2 * NCHUNK, CH)

    cnt = _get_sc_degree()(dst_deg, jnp.zeros((N,), f32), jnp.ones((CH,), f32))
    cnt2 = cnt.reshape(N, 1)

    didx3 = domain_idx.reshape(NB, BLK, 1)
    w1e = W1[:EMB]
    w1x = W1[EMB:]

    g1, dinv = _tc_a(x, didx3, cnt2, emb, w1e, w1x)

    p = _get_sc_scatter()(g1, src_p, dst_p)

    g2 = _tc_b(p[0], p[1], g1, dinv, b1.reshape(1, HID), A1, B1)

    q = _get_sc_scatter()(g2, src_p, dst_p)

    return _tc_c(q[0], q[1], g2, dinv, W2, b2.reshape(1, OUT), A2, B2)


# speedup vs baseline: 1.0361x; 1.0057x over previous
import functools

import jax
import jax.numpy as jnp
from jax import lax
from jax.experimental import pallas as pl
from jax.experimental.pallas import tpu as pltpu
from jax.experimental.pallas import tpu_sc as plsc

N = 10000
E = 320000
DF = 128
EMB = 4096
HID = 32
OUT = 5
RNK = 8

NB = 1
BLK = N // NB

NC = 2
NS = 16
NW = NC * NS
CH = 1000
NCHUNK = 10


@functools.cache
def _get_sc_degree():
    mesh = plsc.VectorSubcoreMesh(core_axis_name="c", subcore_axis_name="s",
                                  num_cores=NC, num_subcores=NS)
    return pl.kernel(
        _sc_degree_body,
        out_type=jax.ShapeDtypeStruct((NC, N), jnp.float32),
        mesh=mesh,
        scratch_types=[
            pltpu.VMEM((NCHUNK, CH), jnp.int32),
            pltpu.VMEM((CH,), jnp.float32),
            pltpu.VMEM_SHARED((N,), jnp.float32),
            pltpu.SemaphoreType.DMA,
        ],
        compiler_params=pltpu.CompilerParams(use_tc_tiling_on_sc=False),
    )


def _sc_degree_body(dst_hbm, zeros_hbm, ones_hbm, out_hbm, dst_v, ones_v, acc, sem):
    cid = lax.axis_index("c")
    sid = lax.axis_index("s")
    wid = cid * NS + sid

    @pl.when(sid == 0)
    def _():
        pltpu.sync_copy(zeros_hbm, acc)
    plsc.subcore_barrier()

    pltpu.sync_copy(ones_hbm, ones_v)
    pltpu.sync_copy(dst_hbm.at[wid], dst_v)

    def body(j, carry):
        pltpu.sync_copy(ones_v, acc.at[dst_v.at[j]], add=True)
        return carry

    lax.fori_loop(0, NCHUNK, body, 0)

    plsc.subcore_barrier()

    @pl.when(sid == 0)
    def _():
        pltpu.sync_copy(acc, out_hbm.at[cid])


@functools.cache
def _get_sc_scatter():
    mesh = plsc.VectorSubcoreMesh(core_axis_name="c", subcore_axis_name="s",
                                  num_cores=NC, num_subcores=NS)
    return pl.kernel(
        _sc_scatter_body,
        out_type=jax.ShapeDtypeStruct((NC, N, HID), jnp.float32),
        mesh=mesh,
        scratch_types=[
            pltpu.VMEM((NCHUNK, CH), jnp.int32),
            pltpu.VMEM((NCHUNK, CH), jnp.int32),
            pltpu.VMEM((CH, HID), jnp.float32),
            pltpu.VMEM((CH, HID), jnp.float32),
            pltpu.VMEM_SHARED((N, HID), jnp.float32),
            pltpu.SemaphoreType.DMA,
            pltpu.SemaphoreType.DMA,
        ],
        compiler_params=pltpu.CompilerParams(use_tc_tiling_on_sc=False),
    )


def _sc_scatter_body(g_hbm, src_hbm, dst_hbm, out_hbm, src_v, dst_v,
                     rows0, rows1, acc, sem0, sem1):
    cid = lax.axis_index("c")
    sid = lax.axis_index("s")
    wid = cid * NS + sid

    @pl.when(sid == 0)
    def _():
        pltpu.sync_copy(g_hbm, acc)
    plsc.subcore_barrier()

    pltpu.sync_copy(src_hbm.at[wid], src_v)
    pltpu.sync_copy(dst_hbm.at[wid], dst_v)

    pltpu.async_copy(g_hbm.at[src_v.at[0]], rows0, sem0)

    def body(i, carry):
        j = 2 * i
        pltpu.async_copy(g_hbm.at[src_v.at[j + 1]], rows1, sem1)
        pltpu.make_async_copy(g_hbm.at[src_v.at[j]], rows0, sem0).wait()
        pltpu.sync_copy(rows0, acc.at[dst_v.at[j]], add=True)

        @pl.when(i < NCHUNK // 2 - 1)
        def _():
            pltpu.async_copy(g_hbm.at[src_v.at[j + 2]], rows0, sem0)

        pltpu.make_async_copy(g_hbm.at[src_v.at[j + 1]], rows1, sem1).wait()
        pltpu.sync_copy(rows1, acc.at[dst_v.at[j + 1]], add=True)
        return carry

    lax.fori_loop(0, NCHUNK // 2, body, 0)
    plsc.subcore_barrier()

    @pl.when(sid == 0)
    def _():
        pltpu.sync_copy(acc, out_hbm.at[cid])


def _tc_a_body(x_ref, didx_ref, cnt0_ref, cnt1_ref, emb_ref, w1e_ref, w1x_ref,
               g1_ref, dinv_ref):
    embw = jnp.dot(emb_ref[...], w1e_ref[...],
                   preferred_element_type=jnp.float32)
    d = didx_ref[0]
    e0 = embw[0:1, :]
    e1 = embw[1:2, :]
    e2 = embw[2:3, :]
    embedded = jnp.where(d == 0, e0, jnp.where(d == 1, e1, e2))
    hx = jnp.dot(x_ref[...], w1x_ref[...],
                 preferred_element_type=jnp.float32)
    dinv = lax.rsqrt(cnt0_ref[...] + cnt1_ref[...] + 1.0)
    g1_ref[...] = dinv * (embedded + hx)
    dinv_ref[...] = dinv


def _tc_a(x, didx3, cnt0, cnt1, emb_raw, w1e, w1x):
    return pl.pallas_call(
        _tc_a_body,
        grid=(NB,),
        in_specs=[
            pl.BlockSpec((BLK, DF), lambda i: (i, 0)),
            pl.BlockSpec((1, BLK, 1), lambda i: (i, 0, 0)),
            pl.BlockSpec((BLK, 1), lambda i: (i, 0)),
            pl.BlockSpec((BLK, 1), lambda i: (i, 0)),
            pl.BlockSpec((3, EMB), lambda i: (0, 0)),
            pl.BlockSpec((EMB, HID), lambda i: (0, 0)),
            pl.BlockSpec((DF, HID), lambda i: (0, 0)),
        ],
        out_specs=[
            pl.BlockSpec((BLK, HID), lambda i: (i, 0)),
            pl.BlockSpec((BLK, 1), lambda i: (i, 0)),
        ],
        out_shape=[
            jax.ShapeDtypeStruct((N, HID), jnp.float32),
            jax.ShapeDtypeStruct((N, 1), jnp.float32),
        ],
    )(x, didx3, cnt0, cnt1, emb_raw, w1e, w1x)


def _tc_b_body(p0_ref, p1_ref, g1_ref, dinv_ref, b1_ref, a1_ref, b1m_ref,
               g2_ref):
    dinv = dinv_ref[...]
    s = (p0_ref[...] + p1_ref[...] - g1_ref[...]) * dinv + b1_ref[...]
    m1 = jnp.dot(a1_ref[...], b1m_ref[...],
                 preferred_element_type=jnp.float32) * (1.0 / RNK)
    h = jnp.maximum(jnp.dot(s, m1, preferred_element_type=jnp.float32), 0.0)
    g2_ref[...] = dinv * h


def _tc_b(p0, p1, g1, dinv, b1r, A1, B1):
    return pl.pallas_call(
        _tc_b_body,
        grid=(NB,),
        in_specs=[
            pl.BlockSpec((BLK, HID), lambda i: (i, 0)),
            pl.BlockSpec((BLK, HID), lambda i: (i, 0)),
            pl.BlockSpec((BLK, HID), lambda i: (i, 0)),
            pl.BlockSpec((BLK, 1), lambda i: (i, 0)),
            pl.BlockSpec((1, HID), lambda i: (0, 0)),
            pl.BlockSpec((HID, RNK), lambda i: (0, 0)),
            pl.BlockSpec((RNK, HID), lambda i: (0, 0)),
        ],
        out_specs=pl.BlockSpec((BLK, HID), lambda i: (i, 0)),
        out_shape=jax.ShapeDtypeStruct((N, HID), jnp.float32),
    )(p0, p1, g1, dinv, b1r, A1, B1)


def _tc_c_body(q0_ref, q1_ref, g2_ref, dinv_ref, w2_ref, b2_ref, a2_ref,
               b2m_ref, out_ref):
    s2 = (q0_ref[...] + q1_ref[...] - g2_ref[...]) * dinv_ref[...]
    ab2 = jnp.dot(a2_ref[...], b2m_ref[...],
                  preferred_element_type=jnp.float32) * (1.0 / RNK)
    m2 = jnp.dot(w2_ref[...], ab2, preferred_element_type=jnp.float32)
    c2 = jnp.dot(b2_ref[...], ab2, preferred_element_type=jnp.float32)
    y = jnp.dot(s2, m2, preferred_element_type=jnp.float32) + c2
    m = jnp.max(y, axis=1, keepdims=True)
    z = y - m
    lse = jnp.log(jnp.sum(jnp.exp(z), axis=1, keepdims=True))
    out_ref[...] = z - lse


def _tc_c(q0, q1, g2, dinv, W2, b2r, A2, B2):
    return pl.pallas_call(
        _tc_c_body,
        grid=(NB,),
        in_specs=[
            pl.BlockSpec((BLK, HID), lambda i: (i, 0)),
            pl.BlockSpec((BLK, HID), lambda i: (i, 0)),
            pl.BlockSpec((BLK, HID), lambda i: (i, 0)),
            pl.BlockSpec((BLK, 1), lambda i: (i, 0)),
            pl.BlockSpec((HID, OUT), lambda i: (0, 0)),
            pl.BlockSpec((1, OUT), lambda i: (0, 0)),
            pl.BlockSpec((OUT, RNK), lambda i: (0, 0)),
            pl.BlockSpec((RNK, OUT), lambda i: (0, 0)),
        ],
        out_specs=pl.BlockSpec((BLK, OUT), lambda i: (i, 0)),
        out_shape=jax.ShapeDtypeStruct((N, OUT), jnp.float32),
    )(q0, q1, g2, dinv, W2, b2r, A2, B2)


def kernel(x, edge_index, domain_idx, emb, W1, b1, A1, B1, W2, b2, A2, B2):
    f32 = jnp.float32
    ei = edge_index.reshape(2, NW, NCHUNK, CH)
    src_p = ei[0]
    dst_p = ei[1]
    cnt = _get_sc_degree()(dst_p, jnp.zeros((N,), f32), jnp.ones((CH,), f32))

    didx3 = domain_idx.reshape(NB, BLK, 1)
    w1e = W1[:EMB]
    w1x = W1[EMB:]

    g1, dinv = _tc_a(x, didx3, cnt[0].reshape(N, 1), cnt[1].reshape(N, 1),
                     emb, w1e, w1x)

    p = _get_sc_scatter()(g1, src_p, dst_p)

    g2 = _tc_b(p[0], p[1], g1, dinv, b1.reshape(1, HID), A1, B1)

    q = _get_sc_scatter()(g2, src_p, dst_p)

    return _tc_c(q[0], q[1], g2, dinv, W2, b2.reshape(1, OUT), A2, B2)
